# baseline (speedup 1.0000x reference)
import functools

import jax
import jax.numpy as jnp
import numpy as np
from jax import lax
from jax.experimental import pallas as pl
from jax.experimental.pallas import tpu as pltpu
from jax.experimental.pallas import tpu_sc as plsc

N = 10000
E = 320000
D1 = 128
D2 = 128
NC, NS = 2, 16
CH = 128
EP = 327680
K = EP // (NC * NS * CH)
NACC = 10112
RPT = NACC // NS
NDEG = 10240
DPT = NDEG // NS
RB = 2000

def _mesh():
    return plsc.VectorSubcoreMesh(
        core_axis_name="c", subcore_axis_name="s", num_cores=NC, num_subcores=NS
    )


@functools.cache
def _make_sc_degree():
    return functools.partial(
        pl.kernel,
        out_type=jax.ShapeDtypeStruct((NC, NDEG), jnp.float32),
        mesh=_mesh(),
        scratch_types=[
            pltpu.VMEM((K, CH), jnp.int32),
            pltpu.VMEM((CH,), jnp.float32),
            pltpu.VMEM_SHARED((NDEG,), jnp.float32),
            pltpu.SemaphoreType.DMA,
        ],
    )(_sc_degree_body)


def _sc_degree_body(dstp_hbm, zeros_hbm, out_hbm, didx_v, ones_v, deg_sh, sem):
    cid = lax.axis_index("c")
    sid = lax.axis_index("s")
    for i in range(CH // 16):
        ones_v[pl.ds(i * 16, 16)] = jnp.full((16,), 1.0, jnp.float32)
    pltpu.sync_copy(zeros_hbm.at[pl.ds(sid * DPT, DPT)],
                    deg_sh.at[pl.ds(sid * DPT, DPT)])
    pltpu.sync_copy(dstp_hbm.at[cid * NS + sid], didx_v)
    plsc.subcore_barrier()

    def body(k, carry):
        pltpu.async_copy(ones_v, deg_sh.at[didx_v.at[k]], sem, add=True)
        return carry

    lax.fori_loop(0, K, body, 0)

    def drain(k, carry):
        pltpu.make_async_copy(ones_v, deg_sh.at[didx_v.at[0]], sem).wait()
        return carry

    lax.fori_loop(0, K, drain, 0)
    plsc.subcore_barrier()
    pltpu.sync_copy(deg_sh.at[pl.ds(sid * DPT, DPT)],
                    out_hbm.at[cid, pl.ds(sid * DPT, DPT)])


@functools.cache
def _make_sc_scatter(D):
    @functools.partial(
        pl.kernel,
        out_type=jax.ShapeDtypeStruct((NC, NACC, D), jnp.float32),
        mesh=_mesh(),
        scratch_types=[
            pltpu.VMEM((K, CH), jnp.int32),
            pltpu.VMEM((K // 2, CH), jnp.int32),
            pltpu.VMEM((2, CH, D), jnp.float32),
            pltpu.VMEM_SHARED((NACC, D), jnp.float32),
            pltpu.SemaphoreType.DMA,
            pltpu.SemaphoreType.DMA,
        ],
        compiler_params=pltpu.CompilerParams(
            use_tc_tiling_on_sc=(D % 128 == 0)),
    )
    def sc_scatter(g_hbm, srcp_hbm, dstp_hbm, zeros_hbm, out_hbm,
                   sidx_v, didx_v, rows_v, acc_sh, gsem, ssem):
        cid = lax.axis_index("c")
        sid = lax.axis_index("s")
        wid = cid * NS + sid
        pltpu.sync_copy(zeros_hbm.at[pl.ds(sid * RPT, RPT)],
                        acc_sh.at[pl.ds(sid * RPT, RPT)])
        pltpu.sync_copy(srcp_hbm.at[wid], sidx_v)
        plsc.subcore_barrier()

        rows_a = rows_v.at[0]
        rows_b = rows_v.at[1]
        K2 = K // 2

        def _wait_scatter(buf):
            pltpu.make_async_copy(buf, acc_sh.at[didx_v.at[0]], ssem).wait()

        def _wait_gather(buf):
            pltpu.make_async_copy(g_hbm.at[sidx_v.at[0]], buf, gsem).wait()

        for h in range(2):
            pltpu.sync_copy(dstp_hbm.at[wid, pl.ds(h * K2, K2)], didx_v)
            pltpu.async_copy(g_hbm.at[sidx_v.at[h * K2]], rows_a, gsem)

            def body(j, carry):
                k0 = 2 * j
                s0 = h * K2 + k0

                @pl.when(j >= 1)
                def _():
                    _wait_scatter(rows_b)

                pltpu.async_copy(g_hbm.at[sidx_v.at[s0 + 1]], rows_b, gsem)
                _wait_gather(rows_a)
                pltpu.async_copy(rows_a, acc_sh.at[didx_v.at[k0]], ssem,
                                 add=True)
                _wait_scatter(rows_a)

                @pl.when(j + 1 < K2 // 2)
                def _():
                    pltpu.async_copy(g_hbm.at[sidx_v.at[s0 + 2]], rows_a,
                                     gsem)

                _wait_gather(rows_b)
                pltpu.async_copy(rows_b, acc_sh.at[didx_v.at[k0 + 1]], ssem,
                                 add=True)
                return carry

            lax.fori_loop(0, K2 // 2, body, 0)
            _wait_scatter(rows_b)
        plsc.subcore_barrier()
        pltpu.sync_copy(acc_sh.at[pl.ds(sid * RPT, RPT)],
                        out_hbm.at[cid, pl.ds(sid * RPT, RPT)])

    return sc_scatter


def _softplus(v):
    return jnp.maximum(v, 0.0) + jnp.log1p(jnp.exp(-jnp.abs(v)))


def _dinv_col(deg_blk):
    return lax.rsqrt(deg_blk[:, 0:1] + deg_blk[:, 1:2] + 1.0)


_LG_N = (10619610.099075997, 11246929.484659938, 5210869.017760828,
         1379496.2658786713, 228235.2154997143, 24165.510665029484,
         1599.042534722041, 60.458333333342054, 0.9999999999998099)
_LG_D = (40320.0, 109584.0, 118124.0, 67284.0, 22449.0, 4536.0, 546.0,
         36.0, 1.0)


def _lgamma1p(y):
    nf = jnp.full_like(y, _LG_N[8])
    df = jnp.full_like(y, _LG_D[8])
    for i in range(7, -1, -1):
        nf = nf * y + _LG_N[i]
        df = df * y + _LG_D[i]
    t = y + 7.5
    return (0.91893853320467274 + (y + 0.5) * jnp.log(t) - t
            + jnp.log(nf) - jnp.log(df))


def _tcr_body(eipr_ref, srcp_ref, dstp_ref):
    srcp_ref[...] = eipr_ref[0]
    dstp_ref[...] = eipr_ref[1]


def _tcr(eipr):
    nw = NC * NS
    return pl.pallas_call(
        _tcr_body,
        grid=(nw,),
        in_specs=[pl.BlockSpec((2, 1, K, CH), lambda r: (0, r, 0, 0))],
        out_specs=[
            pl.BlockSpec((1, K, CH), lambda r: (r, 0, 0)),
            pl.BlockSpec((1, K, CH), lambda r: (r, 0, 0)),
        ],
        out_shape=[
            jax.ShapeDtypeStruct((nw, K, CH), jnp.int32),
            jax.ShapeDtypeStruct((nw, K, CH), jnp.int32),
        ],
    )(eipr)


def _tc0_body(x_ref, w1_ref, h1_ref):
    h1_ref[...] = jnp.dot(x_ref[...], w1_ref[...],
                          preferred_element_type=jnp.float32)


def _tc1_body(h1_ref, deg_ref, g1_ref):
    g1_ref[...] = h1_ref[...] * _dinv_col(deg_ref[...])


def _tc2_body(acc_ref, g1_ref, deg_ref, w2_ref, g2_ref):
    dinv = _dinv_col(deg_ref[...])
    out1 = dinv * (acc_ref[0] + acc_ref[1] + g1_ref[...])
    h1s = _softplus(out1)
    h2 = jnp.dot(h1s, w2_ref[...], preferred_element_type=jnp.float32)
    g2_ref[...] = h2 * dinv


def _tc3_body(acc_ref, g2_ref, deg_ref, z_ref, lbd_ref, kap_ref):
    dinv = _dinv_col(deg_ref[...])
    out2 = dinv * (acc_ref[0] + acc_ref[1] + g2_ref[...])
    h2s = _softplus(out2)
    lbd = h2s[:, 0:64]
    kap = h2s[:, 64:65] + 0.1
    z_ref[...] = lbd * jnp.exp(_lgamma1p(1.0 / kap))
    lbd_ref[...] = lbd
    kap_ref[...] = kap


_GRID = N // RB


def _tc0(x, W1):
    return pl.pallas_call(
        _tc0_body,
        grid=(_GRID,),
        in_specs=[
            pl.BlockSpec((RB, D1), lambda b: (b, 0)),
            pl.BlockSpec((D1, D1), lambda b: (0, 0)),
        ],
        out_specs=pl.BlockSpec((RB, D1), lambda b: (b, 0)),
        out_shape=jax.ShapeDtypeStruct((N, D1), jnp.float32),
    )(x, W1)


def _tc1(h1, degT):
    return pl.pallas_call(
        _tc1_body,
        grid=(_GRID,),
        in_specs=[
            pl.BlockSpec((RB, D1), lambda b: (b, 0)),
            pl.BlockSpec((RB, 2), lambda b: (b, 0)),
        ],
        out_specs=pl.BlockSpec((RB, D1), lambda b: (b, 0)),
        out_shape=jax.ShapeDtypeStruct((N, D1), jnp.float32),
    )(h1, degT)


def _tc2(acc1, g1, degT, W2p):
    return pl.pallas_call(
        _tc2_body,
        grid=(_GRID,),
        in_specs=[
            pl.BlockSpec((NC, RB, D1), lambda b: (0, b, 0)),
            pl.BlockSpec((RB, D1), lambda b: (b, 0)),
            pl.BlockSpec((RB, 2), lambda b: (b, 0)),
            pl.BlockSpec((D1, D2), lambda b: (0, 0)),
        ],
        out_specs=pl.BlockSpec((RB, D2), lambda b: (b, 0)),
        out_shape=jax.ShapeDtypeStruct((N, D2), jnp.float32),
    )(acc1, g1, degT, W2p)


def _tc3(acc2, g2, degT):
    return pl.pallas_call(
        _tc3_body,
        grid=(_GRID,),
        in_specs=[
            pl.BlockSpec((NC, RB, D2), lambda b: (0, b, 0)),
            pl.BlockSpec((RB, D2), lambda b: (b, 0)),
            pl.BlockSpec((RB, 2), lambda b: (b, 0)),
        ],
        out_specs=[
            pl.BlockSpec((RB, 64), lambda b: (b, 0)),
            pl.BlockSpec((RB, 64), lambda b: (b, 0)),
            pl.BlockSpec((RB, 1), lambda b: (b, 0)),
        ],
        out_shape=[
            jax.ShapeDtypeStruct((N, 64), jnp.float32),
            jax.ShapeDtypeStruct((N, 64), jnp.float32),
            jax.ShapeDtypeStruct((N, 1), jnp.float32),
        ],
    )(acc2, g2, degT)


def kernel(x, edge_index, W1, W2, mask_rate=0):
    spread = np.arange(EP - E, dtype=np.int32) % 16
    pad_blk = jnp.asarray(np.stack([spread, N + spread]))
    eip = jnp.concatenate([edge_index, pad_blk], axis=1)
    eipr = eip.reshape(2, NC * NS, K, CH)
    srcp, dstp = _tcr(eipr)
    z1d = jnp.zeros((NDEG,), jnp.float32)
    zd1 = jnp.zeros((NACC, D1), jnp.float32)
    W2p = jnp.pad(W2, ((0, 0), (0, D2 - W2.shape[1])))

    h1 = _tc0(x, W1)
    deg2 = _make_sc_degree()(dstp, z1d)
    degT = deg2.T[:N]
    g1 = _tc1(h1, degT)
    acc1 = _make_sc_scatter(D1)(g1, srcp, dstp, zd1)
    g2 = _tc2(acc1, g1, degT, W2p)
    acc2 = _make_sc_scatter(D2)(g2, srcp, dstp, zd1)
    z, lbd, kap = _tc3(acc2, g2, degT)
    return (z, lbd, kap)


# speedup vs baseline: 1.0012x; 1.0012x over previous
import functools

import jax
import jax.numpy as jnp
import numpy as np
from jax import lax
from jax.experimental import pallas as pl
from jax.experimental.pallas import tpu as pltpu
from jax.experimental.pallas import tpu_sc as plsc

N = 10000
E = 320000
D1 = 128
D2 = 128
DS2 = 80
NC, NS = 2, 16
CH = 128
EP = 327680
K = EP // (NC * NS * CH)
NACC = 10112
RPT = NACC // NS
NDEG = 10240
DPT = NDEG // NS
RB = 2000

def _mesh():
    return plsc.VectorSubcoreMesh(
        core_axis_name="c", subcore_axis_name="s", num_cores=NC, num_subcores=NS
    )


@functools.cache
def _make_sc_degree():
    return functools.partial(
        pl.kernel,
        out_type=jax.ShapeDtypeStruct((NC, NDEG), jnp.float32),
        mesh=_mesh(),
        scratch_types=[
            pltpu.VMEM((K, CH), jnp.int32),
            pltpu.VMEM((CH,), jnp.float32),
            pltpu.VMEM_SHARED((NDEG,), jnp.float32),
            pltpu.SemaphoreType.DMA,
        ],
    )(_sc_degree_body)


def _sc_degree_body(dstp_hbm, zeros_hbm, out_hbm, didx_v, ones_v, deg_sh, sem):
    cid = lax.axis_index("c")
    sid = lax.axis_index("s")
    for i in range(CH // 16):
        ones_v[pl.ds(i * 16, 16)] = jnp.full((16,), 1.0, jnp.float32)
    pltpu.sync_copy(zeros_hbm.at[pl.ds(sid * DPT, DPT)],
                    deg_sh.at[pl.ds(sid * DPT, DPT)])
    pltpu.sync_copy(dstp_hbm.at[cid * NS + sid], didx_v)
    plsc.subcore_barrier()

    def body(k, carry):
        pltpu.async_copy(ones_v, deg_sh.at[didx_v.at[k]], sem, add=True)
        return carry

    lax.fori_loop(0, K, body, 0)

    def drain(k, carry):
        pltpu.make_async_copy(ones_v, deg_sh.at[didx_v.at[0]], sem).wait()
        return carry

    lax.fori_loop(0, K, drain, 0)
    plsc.subcore_barrier()
    pltpu.sync_copy(deg_sh.at[pl.ds(sid * DPT, DPT)],
                    out_hbm.at[cid, pl.ds(sid * DPT, DPT)])


@functools.cache
def _make_sc_scatter(DG, DS):
    @functools.partial(
        pl.kernel,
        out_type=jax.ShapeDtypeStruct((NC, NACC, DS), jnp.float32),
        mesh=_mesh(),
        scratch_types=[
            pltpu.VMEM((K, CH), jnp.int32),
            pltpu.VMEM((K // 2, CH), jnp.int32),
            pltpu.VMEM((2, CH, DS), jnp.float32),
            pltpu.VMEM_SHARED((NACC, DS), jnp.float32),
            pltpu.SemaphoreType.DMA,
            pltpu.SemaphoreType.DMA,
        ],
        compiler_params=pltpu.CompilerParams(
            use_tc_tiling_on_sc=(DS % 128 == 0)),
    )
    def sc_scatter(g_hbm, srcp_hbm, dstp_hbm, zeros_hbm, out_hbm,
                   sidx_v, didx_v, rows_v, acc_sh, gsem, ssem):
        cid = lax.axis_index("c")
        sid = lax.axis_index("s")
        wid = cid * NS + sid
        pltpu.sync_copy(zeros_hbm.at[pl.ds(sid * RPT, RPT)],
                        acc_sh.at[pl.ds(sid * RPT, RPT)])
        pltpu.sync_copy(srcp_hbm.at[wid], sidx_v)
        plsc.subcore_barrier()

        rows_a = rows_v.at[0]
        rows_b = rows_v.at[1]
        K2 = K // 2

        def _gather_src(kk):
            if DS == DG:
                return g_hbm.at[sidx_v.at[kk]]
            return g_hbm.at[sidx_v.at[kk], pl.ds(0, DS)]

        def _wait_scatter(buf):
            pltpu.make_async_copy(buf, acc_sh.at[didx_v.at[0]], ssem).wait()

        def _wait_gather(buf):
            pltpu.make_async_copy(_gather_src(0), buf, gsem).wait()

        for h in range(2):
            pltpu.sync_copy(dstp_hbm.at[wid, pl.ds(h * K2, K2)], didx_v)
            pltpu.async_copy(_gather_src(h * K2), rows_a, gsem)

            def body(j, carry):
                k0 = 2 * j
                s0 = h * K2 + k0

                @pl.when(j >= 1)
                def _():
                    _wait_scatter(rows_b)

                pltpu.async_copy(_gather_src(s0 + 1), rows_b, gsem)
                _wait_gather(rows_a)
                pltpu.async_copy(rows_a, acc_sh.at[didx_v.at[k0]], ssem,
                                 add=True)
                _wait_scatter(rows_a)

                @pl.when(j + 1 < K2 // 2)
                def _():
                    pltpu.async_copy(_gather_src(s0 + 2), rows_a, gsem)

                _wait_gather(rows_b)
                pltpu.async_copy(rows_b, acc_sh.at[didx_v.at[k0 + 1]], ssem,
                                 add=True)
                return carry

            lax.fori_loop(0, K2 // 2, body, 0)
            _wait_scatter(rows_b)
        plsc.subcore_barrier()
        pltpu.sync_copy(acc_sh.at[pl.ds(sid * RPT, RPT)],
                        out_hbm.at[cid, pl.ds(sid * RPT, RPT)])

    return sc_scatter


def _softplus(v):
    return jnp.maximum(v, 0.0) + jnp.log1p(jnp.exp(-jnp.abs(v)))


def _dinv_col(deg_blk):
    return lax.rsqrt(deg_blk[:, 0:1] + deg_blk[:, 1:2] + 1.0)


_LG_N = (10619610.099075997, 11246929.484659938, 5210869.017760828,
         1379496.2658786713, 228235.2154997143, 24165.510665029484,
         1599.042534722041, 60.458333333342054, 0.9999999999998099)
_LG_D = (40320.0, 109584.0, 118124.0, 67284.0, 22449.0, 4536.0, 546.0,
         36.0, 1.0)


def _lgamma1p(y):
    nf = jnp.full_like(y, _LG_N[8])
    df = jnp.full_like(y, _LG_D[8])
    for i in range(7, -1, -1):
        nf = nf * y + _LG_N[i]
        df = df * y + _LG_D[i]
    t = y + 7.5
    return (0.91893853320467274 + (y + 0.5) * jnp.log(t) - t
            + jnp.log(nf) - jnp.log(df))


def _tcr_body(eipr_ref, srcp_ref, dstp_ref):
    srcp_ref[...] = eipr_ref[0]
    dstp_ref[...] = eipr_ref[1]


def _tcr(eipr):
    nw = NC * NS
    return pl.pallas_call(
        _tcr_body,
        grid=(nw,),
        in_specs=[pl.BlockSpec((2, 1, K, CH), lambda r: (0, r, 0, 0))],
        out_specs=[
            pl.BlockSpec((1, K, CH), lambda r: (r, 0, 0)),
            pl.BlockSpec((1, K, CH), lambda r: (r, 0, 0)),
        ],
        out_shape=[
            jax.ShapeDtypeStruct((nw, K, CH), jnp.int32),
            jax.ShapeDtypeStruct((nw, K, CH), jnp.int32),
        ],
    )(eipr)


def _tc0_body(x_ref, w1_ref, h1_ref):
    h1_ref[...] = jnp.dot(x_ref[...], w1_ref[...],
                          preferred_element_type=jnp.float32)


def _tc1_body(h1_ref, deg_ref, g1_ref):
    g1_ref[...] = h1_ref[...] * _dinv_col(deg_ref[...])


def _tc2_body(acc_ref, g1_ref, deg_ref, w2_ref, g2_ref):
    dinv = _dinv_col(deg_ref[...])
    out1 = dinv * (acc_ref[0] + acc_ref[1] + g1_ref[...])
    h1s = _softplus(out1)
    h2 = jnp.dot(h1s, w2_ref[...], preferred_element_type=jnp.float32)
    g2_ref[...] = h2 * dinv


def _tc3_body(acc_ref, g2_ref, deg_ref, z_ref, lbd_ref, kap_ref):
    dinv = _dinv_col(deg_ref[...])
    out2 = dinv * (acc_ref[0] + acc_ref[1] + g2_ref[...])
    h2s = _softplus(out2)
    lbd = h2s[:, 0:64]
    kap = h2s[:, 64:65] + 0.1
    z_ref[...] = lbd * jnp.exp(_lgamma1p(1.0 / kap))
    lbd_ref[...] = lbd
    kap_ref[...] = kap


_GRID = N // RB


def _tc0(x, W1):
    return pl.pallas_call(
        _tc0_body,
        grid=(_GRID,),
        in_specs=[
            pl.BlockSpec((RB, D1), lambda b: (b, 0)),
            pl.BlockSpec((D1, D1), lambda b: (0, 0)),
        ],
        out_specs=pl.BlockSpec((RB, D1), lambda b: (b, 0)),
        out_shape=jax.ShapeDtypeStruct((N, D1), jnp.float32),
    )(x, W1)


def _tc1(h1, degT):
    return pl.pallas_call(
        _tc1_body,
        grid=(_GRID,),
        in_specs=[
            pl.BlockSpec((RB, D1), lambda b: (b, 0)),
            pl.BlockSpec((RB, 2), lambda b: (b, 0)),
        ],
        out_specs=pl.BlockSpec((RB, D1), lambda b: (b, 0)),
        out_shape=jax.ShapeDtypeStruct((N, D1), jnp.float32),
    )(h1, degT)


def _tc2(acc1, g1, degT, W2p):
    return pl.pallas_call(
        _tc2_body,
        grid=(_GRID,),
        in_specs=[
            pl.BlockSpec((NC, RB, D1), lambda b: (0, b, 0)),
            pl.BlockSpec((RB, D1), lambda b: (b, 0)),
            pl.BlockSpec((RB, 2), lambda b: (b, 0)),
            pl.BlockSpec((D1, D2), lambda b: (0, 0)),
        ],
        out_specs=pl.BlockSpec((RB, D2), lambda b: (b, 0)),
        out_shape=jax.ShapeDtypeStruct((N, D2), jnp.float32),
    )(acc1, g1, degT, W2p)


def _tc3(acc2, g2, degT):
    return pl.pallas_call(
        _tc3_body,
        grid=(_GRID,),
        in_specs=[
            pl.BlockSpec((NC, RB, D2), lambda b: (0, b, 0)),
            pl.BlockSpec((RB, D2), lambda b: (b, 0)),
            pl.BlockSpec((RB, 2), lambda b: (b, 0)),
        ],
        out_specs=[
            pl.BlockSpec((RB, 64), lambda b: (b, 0)),
            pl.BlockSpec((RB, 64), lambda b: (b, 0)),
            pl.BlockSpec((RB, 1), lambda b: (b, 0)),
        ],
        out_shape=[
            jax.ShapeDtypeStruct((N, 64), jnp.float32),
            jax.ShapeDtypeStruct((N, 64), jnp.float32),
            jax.ShapeDtypeStruct((N, 1), jnp.float32),
        ],
    )(acc2, g2, degT)


def kernel(x, edge_index, W1, W2, mask_rate=0):
    spread = np.arange(EP - E, dtype=np.int32) % 16
    pad_blk = jnp.asarray(np.stack([spread, N + spread]))
    eip = jnp.concatenate([edge_index, pad_blk], axis=1)
    eipr = eip.reshape(2, NC * NS, K, CH)
    srcp, dstp = _tcr(eipr)
    z1d = jnp.zeros((NDEG,), jnp.float32)
    zd1 = jnp.zeros((NACC, D1), jnp.float32)
    W2p = jnp.pad(W2, ((0, 0), (0, D2 - W2.shape[1])))

    h1 = _tc0(x, W1)
    deg2 = _make_sc_degree()(dstp, z1d)
    degT = deg2.T[:N]
    g1 = _tc1(h1, degT)
    acc1 = _make_sc_scatter(D1, D1)(g1, srcp, dstp, zd1)
    g2 = _tc2(acc1, g1, degT, W2p)
    acc2 = _make_sc_scatter(D2, D2)(g2, srcp, dstp, zd1)
    z, lbd, kap = _tc3(acc2, g2, degT)
    return (z, lbd, kap)


# speedup vs baseline: 1.0095x; 1.0083x over previous
import functools

import jax
import jax.numpy as jnp
import numpy as np
from jax import lax
from jax.experimental import pallas as pl
from jax.experimental.pallas import tpu as pltpu
from jax.experimental.pallas import tpu_sc as plsc

N = 10000
E = 320000
D1 = 128
D2 = 128
DS2 = 80
NC, NS = 2, 16
CH = 128
EP = 327680
K = EP // (NC * NS * CH)
NACC = 10112
RPT = NACC // NS
NDEG = 10240
DPT = NDEG // NS
RB = 2000

def _mesh():
    return plsc.VectorSubcoreMesh(
        core_axis_name="c", subcore_axis_name="s", num_cores=NC, num_subcores=NS
    )


@functools.cache
def _make_sc_degree():
    return functools.partial(
        pl.kernel,
        out_type=jax.ShapeDtypeStruct((NC, NDEG), jnp.float32),
        mesh=_mesh(),
        scratch_types=[
            pltpu.VMEM((K, CH), jnp.int32),
            pltpu.VMEM((CH,), jnp.float32),
            pltpu.VMEM_SHARED((NDEG,), jnp.float32),
            pltpu.SemaphoreType.DMA,
        ],
    )(_sc_degree_body)


def _sc_degree_body(dstp_hbm, zeros_hbm, out_hbm, didx_v, ones_v, deg_sh, sem):
    cid = lax.axis_index("c")
    sid = lax.axis_index("s")
    for i in range(CH // 16):
        ones_v[pl.ds(i * 16, 16)] = jnp.full((16,), 1.0, jnp.float32)
    pltpu.sync_copy(zeros_hbm.at[pl.ds(sid * DPT, DPT)],
                    deg_sh.at[pl.ds(sid * DPT, DPT)])
    pltpu.sync_copy(dstp_hbm.at[cid * NS + sid], didx_v)
    plsc.subcore_barrier()

    def body(k, carry):
        pltpu.async_copy(ones_v, deg_sh.at[didx_v.at[k]], sem, add=True)
        return carry

    lax.fori_loop(0, K, body, 0)

    def drain(k, carry):
        pltpu.make_async_copy(ones_v, deg_sh.at[didx_v.at[0]], sem).wait()
        return carry

    lax.fori_loop(0, K, drain, 0)
    plsc.subcore_barrier()
    pltpu.sync_copy(deg_sh.at[pl.ds(sid * DPT, DPT)],
                    out_hbm.at[cid, pl.ds(sid * DPT, DPT)])


@functools.cache
def _make_sc_scatter(DG, DS):
    @functools.partial(
        pl.kernel,
        out_type=jax.ShapeDtypeStruct((NC, NACC, DS), jnp.float32),
        mesh=_mesh(),
        scratch_types=[
            pltpu.VMEM((K, CH), jnp.int32),
            pltpu.VMEM((K // 2, CH), jnp.int32),
            pltpu.VMEM((2, CH, DS), jnp.float32),
            pltpu.VMEM_SHARED((NACC, DS), jnp.float32),
            pltpu.SemaphoreType.DMA,
            pltpu.SemaphoreType.DMA,
        ],
        compiler_params=pltpu.CompilerParams(
            use_tc_tiling_on_sc=(DS % 128 == 0)),
    )
    def sc_scatter(g_hbm, srcp_hbm, dstp_hbm, zeros_hbm, out_hbm,
                   sidx_v, didx_v, rows_v, acc_sh, gsem, ssem):
        cid = lax.axis_index("c")
        sid = lax.axis_index("s")
        wid = cid * NS + sid
        pltpu.sync_copy(zeros_hbm.at[pl.ds(sid * RPT, RPT)],
                        acc_sh.at[pl.ds(sid * RPT, RPT)])
        pltpu.sync_copy(srcp_hbm.at[wid], sidx_v)
        plsc.subcore_barrier()

        rows_a = rows_v.at[0]
        rows_b = rows_v.at[1]
        K2 = K // 2

        def _gather_src(kk):
            if DS == DG:
                return g_hbm.at[sidx_v.at[kk]]
            return g_hbm.at[sidx_v.at[kk], pl.ds(0, DS)]

        def _wait_scatter(buf):
            pltpu.make_async_copy(buf, acc_sh.at[didx_v.at[0]], ssem).wait()

        def _wait_gather(buf):
            pltpu.make_async_copy(_gather_src(0), buf, gsem).wait()

        for h in range(2):
            pltpu.sync_copy(dstp_hbm.at[wid, pl.ds(h * K2, K2)], didx_v)
            pltpu.async_copy(_gather_src(h * K2), rows_a, gsem)

            def body(j, carry):
                k0 = 2 * j
                s0 = h * K2 + k0

                @pl.when(j >= 1)
                def _():
                    _wait_scatter(rows_b)

                pltpu.async_copy(_gather_src(s0 + 1), rows_b, gsem)
                _wait_gather(rows_a)
                pltpu.async_copy(rows_a, acc_sh.at[didx_v.at[k0]], ssem,
                                 add=True)
                _wait_scatter(rows_a)

                @pl.when(j + 1 < K2 // 2)
                def _():
                    pltpu.async_copy(_gather_src(s0 + 2), rows_a, gsem)

                _wait_gather(rows_b)
                pltpu.async_copy(rows_b, acc_sh.at[didx_v.at[k0 + 1]], ssem,
                                 add=True)
                return carry

            lax.fori_loop(0, K2 // 2, body, 0)
            _wait_scatter(rows_b)
        plsc.subcore_barrier()
        pltpu.sync_copy(acc_sh.at[pl.ds(sid * RPT, RPT)],
                        out_hbm.at[cid, pl.ds(sid * RPT, RPT)])

    return sc_scatter


def _softplus(v):
    return jnp.maximum(v, 0.0) + jnp.log1p(jnp.exp(-jnp.abs(v)))


def _dinv_col(deg_blk):
    return lax.rsqrt(deg_blk[:, 0:1] + deg_blk[:, 1:2] + 1.0)


_LG_N = (10619610.099075997, 11246929.484659938, 5210869.017760828,
         1379496.2658786713, 228235.2154997143, 24165.510665029484,
         1599.042534722041, 60.458333333342054, 0.9999999999998099)
_LG_D = (40320.0, 109584.0, 118124.0, 67284.0, 22449.0, 4536.0, 546.0,
         36.0, 1.0)


def _lgamma1p(y):
    nf = jnp.full_like(y, _LG_N[8])
    df = jnp.full_like(y, _LG_D[8])
    for i in range(7, -1, -1):
        nf = nf * y + _LG_N[i]
        df = df * y + _LG_D[i]
    t = y + 7.5
    return (0.91893853320467274 + (y + 0.5) * jnp.log(t) - t
            + jnp.log(nf) - jnp.log(df))


def _tcr_body(eipr_ref, srcp_ref, dstp_ref):
    srcp_ref[...] = eipr_ref[0]
    dstp_ref[...] = eipr_ref[1]


def _tcr(eipr):
    nw = NC * NS
    return pl.pallas_call(
        _tcr_body,
        grid=(nw,),
        in_specs=[pl.BlockSpec((2, 1, K, CH), lambda r: (0, r, 0, 0))],
        out_specs=[
            pl.BlockSpec((1, K, CH), lambda r: (r, 0, 0)),
            pl.BlockSpec((1, K, CH), lambda r: (r, 0, 0)),
        ],
        out_shape=[
            jax.ShapeDtypeStruct((nw, K, CH), jnp.int32),
            jax.ShapeDtypeStruct((nw, K, CH), jnp.int32),
        ],
    )(eipr)


def _tc0_body(x_ref, w1_ref, h1_ref):
    h1_ref[...] = jnp.dot(x_ref[...], w1_ref[...],
                          preferred_element_type=jnp.float32)


def _tc1_body(h1_ref, deg_ref, g1_ref):
    g1_ref[...] = h1_ref[...] * _dinv_col(deg_ref[...])


def _tc2_body(acc_ref, g1_ref, deg_ref, w2_ref, g2_ref):
    dinv = _dinv_col(deg_ref[...])
    out1 = dinv * (acc_ref[0] + acc_ref[1] + g1_ref[...])
    h1s = _softplus(out1)
    h2 = jnp.dot(h1s, w2_ref[...], preferred_element_type=jnp.float32)
    g2_ref[...] = h2 * dinv


def _tc3_body(acc_ref, g2_ref, deg_ref, z_ref, lbd_ref, kap_ref):
    dinv = _dinv_col(deg_ref[...])
    out2 = dinv * (acc_ref[0] + acc_ref[1] + g2_ref[...])
    h2s = _softplus(out2)
    lbd = h2s[:, 0:64]
    kap = h2s[:, 64:65] + 0.1
    z_ref[...] = lbd * jnp.exp(_lgamma1p(1.0 / kap))
    lbd_ref[...] = lbd
    kap_ref[...] = kap


_GRID = N // RB


def _tc0(x, W1):
    return pl.pallas_call(
        _tc0_body,
        grid=(_GRID,),
        in_specs=[
            pl.BlockSpec((RB, D1), lambda b: (b, 0)),
            pl.BlockSpec((D1, D1), lambda b: (0, 0)),
        ],
        out_specs=pl.BlockSpec((RB, D1), lambda b: (b, 0)),
        out_shape=jax.ShapeDtypeStruct((N, D1), jnp.float32),
    )(x, W1)


def _tc1(h1, degT):
    return pl.pallas_call(
        _tc1_body,
        grid=(_GRID,),
        in_specs=[
            pl.BlockSpec((RB, D1), lambda b: (b, 0)),
            pl.BlockSpec((RB, 2), lambda b: (b, 0)),
        ],
        out_specs=pl.BlockSpec((RB, D1), lambda b: (b, 0)),
        out_shape=jax.ShapeDtypeStruct((N, D1), jnp.float32),
    )(h1, degT)


def _tc2(acc1, g1, degT, W2p):
    return pl.pallas_call(
        _tc2_body,
        grid=(_GRID,),
        in_specs=[
            pl.BlockSpec((NC, RB, D1), lambda b: (0, b, 0)),
            pl.BlockSpec((RB, D1), lambda b: (b, 0)),
            pl.BlockSpec((RB, 2), lambda b: (b, 0)),
            pl.BlockSpec((D1, D2), lambda b: (0, 0)),
        ],
        out_specs=pl.BlockSpec((RB, D2), lambda b: (b, 0)),
        out_shape=jax.ShapeDtypeStruct((N, D2), jnp.float32),
    )(acc1, g1, degT, W2p)


def _tc3(acc2, g2, degT):
    return pl.pallas_call(
        _tc3_body,
        grid=(_GRID,),
        in_specs=[
            pl.BlockSpec((NC, RB, D2), lambda b: (0, b, 0)),
            pl.BlockSpec((RB, D2), lambda b: (b, 0)),
            pl.BlockSpec((RB, 2), lambda b: (b, 0)),
        ],
        out_specs=[
            pl.BlockSpec((RB, 64), lambda b: (b, 0)),
            pl.BlockSpec((RB, 64), lambda b: (b, 0)),
            pl.BlockSpec((RB, 1), lambda b: (b, 0)),
        ],
        out_shape=[
            jax.ShapeDtypeStruct((N, 64), jnp.float32),
            jax.ShapeDtypeStruct((N, 64), jnp.float32),
            jax.ShapeDtypeStruct((N, 1), jnp.float32),
        ],
    )(acc2, g2, degT)


def kernel(x, edge_index, W1, W2, mask_rate=0):
    spread = np.arange(EP - E, dtype=np.int32) % 16
    srcp = jnp.concatenate([edge_index[0], jnp.asarray(spread)]).reshape(
        NC * NS, K, CH)
    dstp = jnp.concatenate([edge_index[1], jnp.asarray(N + spread)]).reshape(
        NC * NS, K, CH)
    z1d = jnp.zeros((NDEG,), jnp.float32)
    zd1 = jnp.zeros((NACC, D1), jnp.float32)
    W2p = jnp.pad(W2, ((0, 0), (0, D2 - W2.shape[1])))

    h1 = _tc0(x, W1)
    deg2 = _make_sc_degree()(dstp, z1d)
    degT = deg2.T[:N]
    g1 = _tc1(h1, degT)
    acc1 = _make_sc_scatter(D1, D1)(g1, srcp, dstp, zd1)
    g2 = _tc2(acc1, g1, degT, W2p)
    acc2 = _make_sc_scatter(D2, D2)(g2, srcp, dstp, zd1)
    z, lbd, kap = _tc3(acc2, g2, degT)
    return (z, lbd, kap)


# speedup vs baseline: 1.0117x; 1.0022x over previous
import functools

import jax
import jax.numpy as jnp
import numpy as np
from jax import lax
from jax.experimental import pallas as pl
from jax.experimental.pallas import tpu as pltpu
from jax.experimental.pallas import tpu_sc as plsc

N = 10000
E = 320000
D1 = 128
D2 = 128
NC, NS = 2, 16
CH = 128
EP = 327680
K = EP // (NC * NS * CH)
NACC = 10112
RPT = NACC // NS
NDEG = 10240
DPT = NDEG // NS
RB = 2000

def _mesh():
    return plsc.VectorSubcoreMesh(
        core_axis_name="c", subcore_axis_name="s", num_cores=NC, num_subcores=NS
    )


@functools.cache
def _make_sc_degree():
    return functools.partial(
        pl.kernel,
        out_type=jax.ShapeDtypeStruct((NC, NDEG), jnp.float32),
        mesh=_mesh(),
        scratch_types=[
            pltpu.VMEM((K, CH), jnp.int32),
            pltpu.VMEM((CH,), jnp.float32),
            pltpu.VMEM_SHARED((NDEG,), jnp.float32),
            pltpu.SemaphoreType.DMA,
        ],
    )(_sc_degree_body)


def _sc_degree_body(dstp_hbm, zeros_hbm, out_hbm, didx_v, ones_v, deg_sh, sem):
    cid = lax.axis_index("c")
    sid = lax.axis_index("s")
    for i in range(CH // 16):
        ones_v[pl.ds(i * 16, 16)] = jnp.full((16,), 1.0, jnp.float32)
    pltpu.sync_copy(zeros_hbm.at[pl.ds(sid * DPT, DPT)],
                    deg_sh.at[pl.ds(sid * DPT, DPT)])
    pltpu.sync_copy(dstp_hbm.at[cid * NS + sid], didx_v)
    plsc.subcore_barrier()

    def body(k, carry):
        pltpu.async_copy(ones_v, deg_sh.at[didx_v.at[k]], sem, add=True)
        return carry

    lax.fori_loop(0, K, body, 0)

    def drain(k, carry):
        pltpu.make_async_copy(ones_v, deg_sh.at[didx_v.at[0]], sem).wait()
        return carry

    lax.fori_loop(0, K, drain, 0)
    plsc.subcore_barrier()
    pltpu.sync_copy(deg_sh.at[pl.ds(sid * DPT, DPT)],
                    out_hbm.at[cid, pl.ds(sid * DPT, DPT)])


@functools.cache
def _make_sc_scatter(DG, DS):
    @functools.partial(
        pl.kernel,
        out_type=jax.ShapeDtypeStruct((NC, NACC, DS), jnp.float32),
        mesh=_mesh(),
        scratch_types=[
            pltpu.VMEM((K, CH), jnp.int32),
            pltpu.VMEM((K // 2, CH), jnp.int32),
            pltpu.VMEM((2, CH, DS), jnp.float32),
            pltpu.VMEM_SHARED((NACC, DS), jnp.float32),
            pltpu.SemaphoreType.DMA,
            pltpu.SemaphoreType.DMA,
        ],
        compiler_params=pltpu.CompilerParams(
            use_tc_tiling_on_sc=(DS % 128 == 0)),
    )
    def sc_scatter(g_hbm, srcp_hbm, dstp_hbm, zeros_hbm, out_hbm,
                   sidx_v, didx_v, rows_v, acc_sh, gsem, ssem):
        cid = lax.axis_index("c")
        sid = lax.axis_index("s")
        wid = cid * NS + sid
        pltpu.sync_copy(zeros_hbm.at[pl.ds(sid * RPT, RPT)],
                        acc_sh.at[pl.ds(sid * RPT, RPT)])
        pltpu.sync_copy(srcp_hbm.at[wid], sidx_v)
        plsc.subcore_barrier()

        rows_a = rows_v.at[0]
        rows_b = rows_v.at[1]
        K2 = K // 2

        def _gather_src(kk):
            if DS == DG:
                return g_hbm.at[sidx_v.at[kk]]
            return g_hbm.at[sidx_v.at[kk], pl.ds(0, DS)]

        def _wait_scatter(buf):
            pltpu.make_async_copy(buf, acc_sh.at[didx_v.at[0]], ssem).wait()

        def _wait_gather(buf):
            pltpu.make_async_copy(_gather_src(0), buf, gsem).wait()

        for h in range(2):
            pltpu.sync_copy(dstp_hbm.at[wid, pl.ds(h * K2, K2)], didx_v)
            pltpu.async_copy(_gather_src(h * K2), rows_a, gsem)

            def body(j, carry):
                k0 = 2 * j
                s0 = h * K2 + k0

                @pl.when(j >= 1)
                def _():
                    _wait_scatter(rows_b)

                pltpu.async_copy(_gather_src(s0 + 1), rows_b, gsem)
                _wait_gather(rows_a)
                pltpu.async_copy(rows_a, acc_sh.at[didx_v.at[k0]], ssem,
                                 add=True)
                _wait_scatter(rows_a)

                @pl.when(j + 1 < K2 // 2)
                def _():
                    pltpu.async_copy(_gather_src(s0 + 2), rows_a, gsem)

                _wait_gather(rows_b)
                pltpu.async_copy(rows_b, acc_sh.at[didx_v.at[k0 + 1]], ssem,
                                 add=True)
                return carry

            lax.fori_loop(0, K2 // 2, body, 0)
            _wait_scatter(rows_b)
        plsc.subcore_barrier()
        pltpu.sync_copy(acc_sh.at[pl.ds(sid * RPT, RPT)],
                        out_hbm.at[cid, pl.ds(sid * RPT, RPT)])

    return sc_scatter


def _softplus(v):
    return jnp.maximum(v, 0.0) + jnp.log1p(jnp.exp(-jnp.abs(v)))


def _dinv_col(deg_blk):
    return lax.rsqrt(deg_blk[:, 0:1] + deg_blk[:, 1:2] + 1.0)


_LG_N = (10619610.099075997, 11246929.484659938, 5210869.017760828,
         1379496.2658786713, 228235.2154997143, 24165.510665029484,
         1599.042534722041, 60.458333333342054, 0.9999999999998099)
_LG_D = (40320.0, 109584.0, 118124.0, 67284.0, 22449.0, 4536.0, 546.0,
         36.0, 1.0)


def _lgamma1p(y):
    nf = jnp.full_like(y, _LG_N[8])
    df = jnp.full_like(y, _LG_D[8])
    for i in range(7, -1, -1):
        nf = nf * y + _LG_N[i]
        df = df * y + _LG_D[i]
    t = y + 7.5
    return (0.91893853320467274 + (y + 0.5) * jnp.log(t) - t
            + jnp.log(nf) - jnp.log(df))


def _tc0_body(x_ref, w1_ref, h1_ref):
    h1_ref[...] = jnp.dot(x_ref[...], w1_ref[...],
                          preferred_element_type=jnp.float32)


def _tc1_body(h1_ref, deg_ref, g1_ref):
    g1_ref[...] = h1_ref[...] * _dinv_col(deg_ref[...])


def _tc2_body(acc_ref, g1_ref, deg_ref, w2_ref, g2_ref):
    dinv = _dinv_col(deg_ref[...])
    out1 = dinv * (acc_ref[0] + acc_ref[1] + g1_ref[...])
    h1s = _softplus(out1)
    h2 = jnp.dot(h1s, w2_ref[...], preferred_element_type=jnp.float32)
    g2_ref[...] = h2 * dinv


def _tc3_body(acc_ref, g2_ref, deg_ref, z_ref, lbd_ref, kap_ref):
    dinv = _dinv_col(deg_ref[...])
    out2 = dinv * (acc_ref[0] + acc_ref[1] + g2_ref[...])
    h2s = _softplus(out2)
    lbd = h2s[:, 0:64]
    kap = h2s[:, 64:65] + 0.1
    z_ref[...] = lbd * jnp.exp(_lgamma1p(1.0 / kap))
    lbd_ref[...] = lbd
    kap_ref[...] = kap


_GRID = N // RB


def _tc0(x, W1):
    return pl.pallas_call(
        _tc0_body,
        grid=(_GRID,),
        in_specs=[
            pl.BlockSpec((RB, D1), lambda b: (b, 0)),
            pl.BlockSpec((D1, D1), lambda b: (0, 0)),
        ],
        out_specs=pl.BlockSpec((RB, D1), lambda b: (b, 0)),
        out_shape=jax.ShapeDtypeStruct((N, D1), jnp.float32),
    )(x, W1)


def _tc1(h1, degT):
    return pl.pallas_call(
        _tc1_body,
        grid=(_GRID,),
        in_specs=[
            pl.BlockSpec((RB, D1), lambda b: (b, 0)),
            pl.BlockSpec((RB, 2), lambda b: (b, 0)),
        ],
        out_specs=pl.BlockSpec((RB, D1), lambda b: (b, 0)),
        out_shape=jax.ShapeDtypeStruct((N, D1), jnp.float32),
    )(h1, degT)


def _tc2(acc1, g1, degT, W2p):
    return pl.pallas_call(
        _tc2_body,
        grid=(_GRID,),
        in_specs=[
            pl.BlockSpec((NC, RB, D1), lambda b: (0, b, 0)),
            pl.BlockSpec((RB, D1), lambda b: (b, 0)),
            pl.BlockSpec((RB, 2), lambda b: (b, 0)),
            pl.BlockSpec((D1, D2), lambda b: (0, 0)),
        ],
        out_specs=pl.BlockSpec((RB, D2), lambda b: (b, 0)),
        out_shape=jax.ShapeDtypeStruct((N, D2), jnp.float32),
    )(acc1, g1, degT, W2p)


def _tc3(acc2, g2, degT):
    return pl.pallas_call(
        _tc3_body,
        grid=(_GRID,),
        in_specs=[
            pl.BlockSpec((NC, RB, D2), lambda b: (0, b, 0)),
            pl.BlockSpec((RB, D2), lambda b: (b, 0)),
            pl.BlockSpec((RB, 2), lambda b: (b, 0)),
        ],
        out_specs=[
            pl.BlockSpec((RB, 64), lambda b: (b, 0)),
            pl.BlockSpec((RB, 64), lambda b: (b, 0)),
            pl.BlockSpec((RB, 1), lambda b: (b, 0)),
        ],
        out_shape=[
            jax.ShapeDtypeStruct((N, 64), jnp.float32),
            jax.ShapeDtypeStruct((N, 64), jnp.float32),
            jax.ShapeDtypeStruct((N, 1), jnp.float32),
        ],
    )(acc2, g2, degT)


def kernel(x, edge_index, W1, W2, mask_rate=0):
    spread = np.arange(EP - E, dtype=np.int32) % 16
    srcp = jnp.concatenate([edge_index[0], jnp.asarray(spread)]).reshape(
        NC * NS, K, CH)
    dstp = jnp.concatenate([edge_index[1], jnp.asarray(N + spread)]).reshape(
        NC * NS, K, CH)
    z1d = jnp.zeros((NDEG,), jnp.float32)
    zd1 = jnp.zeros((NACC, D1), jnp.float32)
    W2p = jnp.pad(W2, ((0, 0), (0, D2 - W2.shape[1])))

    h1 = _tc0(x, W1)
    deg2 = _make_sc_degree()(dstp, z1d)
    degT = deg2.T[:N]
    g1 = _tc1(h1, degT)
    acc1 = _make_sc_scatter(D1, D1)(g1, srcp, dstp, zd1)
    g2 = _tc2(acc1, g1, degT, W2p)
    acc2 = _make_sc_scatter(D2, D2)(g2, srcp, dstp, zd1)
    z, lbd, kap = _tc3(acc2, g2, degT)
    return (z, lbd, kap)
